# baseline (speedup 1.0000x reference)
import functools

import jax
import jax.numpy as jnp
from jax import lax
from jax.experimental import pallas as pl
from jax.experimental.pallas import tpu as pltpu
from jax.experimental.pallas import tpu_sc as plsc

EMB_DIM = 128
G = 128
NG = 2
SUP = G * NG
BLK = 2048
TC_FRAC = 0.335


@functools.lru_cache(maxsize=None)
def _make_sc_lookup(n_idx: int, n_emb: int, d: int):
    info = plsc.get_sparse_core_info()
    nw = info.num_cores * info.num_subcores
    assert n_idx % (nw * 2 * SUP) == 0
    per_w = n_idx // nw
    n_chunks = per_w // SUP
    mesh = plsc.VectorSubcoreMesh(core_axis_name="c", subcore_axis_name="s")

    @functools.partial(
        pl.kernel,
        mesh=mesh,
        out_type=jax.ShapeDtypeStruct((n_idx, d), jnp.float32),
        scratch_types=[
            pltpu.VMEM((n_emb, d), jnp.float32),
            pltpu.VMEM_SHARED((n_emb, d), jnp.float32),
            pltpu.VMEM((2, SUP), jnp.int32),
            pltpu.VMEM((2, SUP, d), jnp.float32),
            pltpu.SemaphoreType.DMA,
            pltpu.SemaphoreType.DMA,
            pltpu.SemaphoreType.DMA,
            pltpu.SemaphoreType.DMA,
            pltpu.SemaphoreType.DMA,
            pltpu.SemaphoreType.DMA,
        ],
    )
    def lookup(table_hbm, idx_hbm, out_hbm, table_v, table_sp, idx_v, rows_v,
               g0, g1, o0, o1, i0sem, i1sem):
        wid = lax.axis_index("s") * info.num_cores + lax.axis_index("c")
        base = wid * per_w
        gsem = (g0, g1)
        osem = (o0, o1)
        isem = (i0sem, i1sem)

        @pl.when(lax.axis_index("s") == 0)
        def _():
            pltpu.sync_copy(table_hbm, table_v)
            pltpu.sync_copy(table_v, table_sp)

        plsc.subcore_barrier()

        def start_idx(i, b):
            iw = lax.rem(i, n_chunks)
            pltpu.async_copy(idx_hbm.at[pl.ds(base + iw * SUP, SUP)],
                             idx_v.at[b], isem[b])

        def wait_idx(i, b):
            iw = lax.rem(i, n_chunks)
            pltpu.make_async_copy(idx_hbm.at[pl.ds(base + iw * SUP, SUP)],
                                  idx_v.at[b], isem[b]).wait()

        def start_gather(i, b):
            for k in range(NG):
                pltpu.async_copy(table_sp.at[idx_v.at[b, pl.ds(k * G, G)]],
                                 rows_v.at[b, pl.ds(k * G, G)], gsem[b])

        def wait_gather(b):
            for k in range(NG):
                pltpu.make_async_copy(
                    table_sp.at[idx_v.at[b, pl.ds(k * G, G)]],
                    rows_v.at[b, pl.ds(k * G, G)], gsem[b]).wait()

        def start_store(i, b):
            pltpu.async_copy(rows_v.at[b],
                             out_hbm.at[pl.ds(base + i * SUP, SUP)],
                             osem[b])

        def wait_store(i, b):
            pltpu.make_async_copy(rows_v.at[b],
                                  out_hbm.at[pl.ds(base + i * SUP, SUP)],
                                  osem[b]).wait()

        start_idx(0, 0)
        wait_idx(0, 0)
        start_gather(0, 0)
        start_idx(1, 1)
        wait_idx(1, 1)
        start_gather(1, 1)
        wait_gather(0)
        start_store(0, 0)
        start_idx(2, 0)

        def body(j, carry):
            i0 = 2 * j
            i1 = i0 + 1
            wait_store(i0 - 2, 0)
            wait_idx(i0, 0)
            start_gather(i0, 0)
            wait_gather(1)
            start_store(i0 - 1, 1)
            start_idx(i1, 1)
            wait_store(i1 - 2, 1)
            wait_idx(i1, 1)
            start_gather(i1, 1)
            wait_gather(0)
            start_store(i0, 0)
            start_idx(i0 + 2, 0)
            return carry

        lax.fori_loop(1, n_chunks // 2, body, 0)

        wait_gather(1)
        start_store(n_chunks - 1, 1)
        wait_idx(n_chunks, 0)
        wait_store(n_chunks - 2, 0)
        wait_store(n_chunks - 1, 1)

    return lookup


@functools.lru_cache(maxsize=None)
def _make_tc_lookup(n_idx: int, n_emb: int, d: int):
    nblk = n_idx // BLK

    def body(idx_ref, table_ref, out_ref):
        ids = idx_ref[0, 0, :]
        onehot = (ids[:, None] == lax.iota(jnp.int32, n_emb)[None, :])
        out_ref[...] = jnp.dot(onehot.astype(jnp.float32), table_ref[...],
                               preferred_element_type=jnp.float32)

    return pl.pallas_call(
        body,
        grid=(nblk,),
        in_specs=[
            pl.BlockSpec((1, 1, BLK), lambda i: (i, 0, 0)),
            pl.BlockSpec((n_emb, d), lambda i: (0, 0)),
        ],
        out_specs=pl.BlockSpec((BLK, d), lambda i: (i, 0)),
        out_shape=jax.ShapeDtypeStruct((n_idx, d), jnp.float32),
    )


def _split_sizes(n: int) -> tuple[int, int]:
    sc_quantum = 32 * 2 * SUP
    n_sc = int(n * (1.0 - TC_FRAC)) // sc_quantum * sc_quantum
    n_tc = n - n_sc
    assert n_tc % BLK == 0
    return n_sc, n_tc


def kernel(idx, x, table):
    del x
    b, l = idx.shape
    n = b * l
    d = table.shape[1]
    n_sc, n_tc = _split_sizes(n)
    idx_flat = idx.reshape(n).astype(jnp.int32)
    table_f = table.astype(jnp.float32)
    out_sc = _make_sc_lookup(n_sc, table.shape[0], d)(
        table_f, idx_flat[:n_sc])
    idx3 = idx_flat[n_sc:].reshape(n_tc // BLK, 1, BLK)
    out_tc = _make_tc_lookup(n_tc, table.shape[0], d)(idx3, table_f)
    out = jnp.concatenate([out_sc, out_tc], axis=0)
    return out.reshape(b, l, d)


# speedup vs baseline: 2.5214x; 2.5214x over previous
import functools

import jax
import jax.numpy as jnp
from jax import lax
from jax.experimental import pallas as pl
from jax.experimental.pallas import tpu as pltpu
from jax.experimental.pallas import tpu_sc as plsc

EMB_DIM = 128
G = 128
NG = 2
SUP = G * NG


@functools.lru_cache(maxsize=None)
def _make_lookup(n_idx: int, n_emb: int, d: int):
    info = plsc.get_sparse_core_info()
    nw = info.num_cores * info.num_subcores
    assert n_idx % (nw * SUP) == 0
    per_w = n_idx // nw
    n_chunks = per_w // SUP
    assert n_chunks % 3 == 1 and n_chunks >= 4
    n_body = (n_chunks - 4) // 3
    mesh = plsc.VectorSubcoreMesh(core_axis_name="c", subcore_axis_name="s")

    @functools.partial(
        pl.kernel,
        mesh=mesh,
        out_type=jax.ShapeDtypeStruct((n_idx, d), jnp.float32),
        scratch_types=[
            pltpu.VMEM((n_emb, d), jnp.float32),
            pltpu.VMEM_SHARED((n_emb, d), jnp.float32),
            pltpu.VMEM((SUP,), jnp.int32),
            pltpu.VMEM((SUP,), jnp.int32),
            pltpu.VMEM((SUP,), jnp.int32),
            pltpu.VMEM((SUP, d), jnp.float32),
            pltpu.VMEM((SUP, d), jnp.float32),
            pltpu.VMEM((SUP, d), jnp.float32),
            pltpu.SemaphoreType.DMA,
            pltpu.SemaphoreType.DMA,
            pltpu.SemaphoreType.DMA,
            pltpu.SemaphoreType.DMA,
            pltpu.SemaphoreType.DMA,
            pltpu.SemaphoreType.DMA,
            pltpu.SemaphoreType.DMA,
            pltpu.SemaphoreType.DMA,
            pltpu.SemaphoreType.DMA,
        ],
    )
    def lookup(table_hbm, idx_hbm, out_hbm, table_v, table_sp,
               idx_b0, idx_b1, idx_b2, rows_b0, rows_b1, rows_b2,
               g0, g1, g2, o0, o1, o2, i0s, i1s, i2s):
        wid = lax.axis_index("s") * info.num_cores + lax.axis_index("c")
        base = wid * per_w
        idx_bufs = (idx_b0, idx_b1, idx_b2)
        rows_bufs = (rows_b0, rows_b1, rows_b2)
        gsem = (g0, g1, g2)
        osem = (o0, o1, o2)
        isem = (i0s, i1s, i2s)

        @pl.when(lax.axis_index("s") == 0)
        def _():
            pltpu.sync_copy(table_hbm, table_v)
            pltpu.sync_copy(table_v, table_sp)

        plsc.subcore_barrier()

        def start_idx(i, b):
            iw = lax.rem(i, n_chunks)
            pltpu.async_copy(idx_hbm.at[pl.ds(base + iw * SUP, SUP)],
                             idx_bufs[b], isem[b])

        def wait_idx(i, b):
            iw = lax.rem(i, n_chunks)
            pltpu.make_async_copy(idx_hbm.at[pl.ds(base + iw * SUP, SUP)],
                                  idx_bufs[b], isem[b]).wait()

        def start_gather(i, b):
            for k in range(NG):
                pltpu.async_copy(table_sp.at[idx_bufs[b].at[pl.ds(k * G, G)]],
                                 rows_bufs[b].at[pl.ds(k * G, G)], gsem[b])

        def wait_gather(b):
            for k in range(NG):
                pltpu.make_async_copy(
                    table_sp.at[idx_bufs[b].at[pl.ds(k * G, G)]],
                    rows_bufs[b].at[pl.ds(k * G, G)], gsem[b]).wait()

        def start_store(i, b):
            pltpu.async_copy(rows_bufs[b],
                             out_hbm.at[pl.ds(base + i * SUP, SUP)],
                             osem[b])

        def wait_store(i, b):
            pltpu.make_async_copy(rows_bufs[b],
                                  out_hbm.at[pl.ds(base + i * SUP, SUP)],
                                  osem[b]).wait()

        start_idx(0, 0)
        start_idx(1, 1)
        start_idx(2, 2)
        wait_idx(0, 0)
        start_gather(0, 0)
        wait_idx(1, 1)
        start_gather(1, 1)
        wait_gather(0)
        start_store(0, 0)
        start_idx(3, 0)
        wait_idx(2, 2)
        start_gather(2, 2)
        wait_gather(1)
        start_store(1, 1)
        start_idx(4, 1)

        def body(m, carry):
            i = 3 * m
            for s in range(3):
                b = s
                bp = (s + 2) % 3
                wait_store(i + s - 3, b)
                wait_idx(i + s, b)
                start_gather(i + s, b)
                wait_gather(bp)
                start_store(i + s - 1, bp)
                start_idx(i + s + 2, bp)
            return carry

        lax.fori_loop(1, n_body + 1, body, 0)

        last = n_chunks - 1
        wait_store(last - 3, 0)
        wait_idx(last, 0)
        start_gather(last, 0)
        wait_gather(2)
        start_store(last - 1, 2)
        wait_gather(0)
        start_store(last, 0)
        wait_idx(n_chunks, 1)
        wait_store(last - 2, 1)
        wait_store(last - 1, 2)
        wait_store(last, 0)

    return lookup


def kernel(idx, x, table):
    del x
    b, l = idx.shape
    n = b * l
    idx_flat = idx.reshape(n).astype(jnp.int32)
    lookup = _make_lookup(n, table.shape[0], table.shape[1])
    out = lookup(table.astype(jnp.float32), idx_flat)
    return out.reshape(b, l, table.shape[1])


# speedup vs baseline: 2.5625x; 1.0163x over previous
import functools

import jax
import jax.numpy as jnp
from jax import lax
from jax.experimental import pallas as pl
from jax.experimental.pallas import tpu as pltpu
from jax.experimental.pallas import tpu_sc as plsc

EMB_DIM = 128
G = 128
NG = 2
SUP = G * NG


@functools.lru_cache(maxsize=None)
def _make_lookup(n_idx: int, n_emb: int, d: int):
    info = plsc.get_sparse_core_info()
    nw = info.num_cores * info.num_subcores
    assert n_idx % (nw * 2 * SUP) == 0
    per_w = n_idx // nw
    n_chunks = per_w // SUP
    mesh = plsc.VectorSubcoreMesh(core_axis_name="c", subcore_axis_name="s")

    @functools.partial(
        pl.kernel,
        mesh=mesh,
        out_type=jax.ShapeDtypeStruct((n_idx, d), jnp.float32),
        scratch_types=[
            pltpu.VMEM((n_emb, d), jnp.float32),
            pltpu.VMEM_SHARED((n_emb, d), jnp.float32),
            pltpu.VMEM((2, SUP), jnp.int32),
            pltpu.VMEM((2, SUP, d), jnp.float32),
            pltpu.SemaphoreType.DMA,
            pltpu.SemaphoreType.DMA,
            pltpu.SemaphoreType.DMA,
            pltpu.SemaphoreType.DMA,
            pltpu.SemaphoreType.DMA,
            pltpu.SemaphoreType.DMA,
        ],
    )
    def lookup(table_hbm, idx_hbm, out_hbm, table_v, table_sp, idx_v, rows_v,
               g0, g1, o0, o1, i0sem, i1sem):
        wid = lax.axis_index("s") * info.num_cores + lax.axis_index("c")
        base = wid * per_w
        gsem = (g0, g1)
        osem = (o0, o1)
        isem = (i0sem, i1sem)

        @pl.when(lax.axis_index("s") == 0)
        def _():
            pltpu.sync_copy(table_hbm, table_v)
            pltpu.sync_copy(table_v, table_sp)

        plsc.subcore_barrier()

        def start_idx(i, b):
            iw = lax.rem(i, n_chunks)
            pltpu.async_copy(idx_hbm.at[pl.ds(base + iw * SUP, SUP)],
                             idx_v.at[b], isem[b])

        def wait_idx(i, b):
            iw = lax.rem(i, n_chunks)
            pltpu.make_async_copy(idx_hbm.at[pl.ds(base + iw * SUP, SUP)],
                                  idx_v.at[b], isem[b]).wait()

        def start_gather(i, b):
            for k in range(NG):
                pltpu.async_copy(table_sp.at[idx_v.at[b, pl.ds(k * G, G)]],
                                 rows_v.at[b, pl.ds(k * G, G)], gsem[b])

        def wait_gather(b):
            for k in range(NG):
                pltpu.make_async_copy(
                    table_sp.at[idx_v.at[b, pl.ds(k * G, G)]],
                    rows_v.at[b, pl.ds(k * G, G)], gsem[b]).wait()

        def start_store(i, b):
            pltpu.async_copy(rows_v.at[b],
                             out_hbm.at[pl.ds(base + i * SUP, SUP)],
                             osem[b])

        def wait_store(i, b):
            pltpu.make_async_copy(rows_v.at[b],
                                  out_hbm.at[pl.ds(base + i * SUP, SUP)],
                                  osem[b]).wait()

        start_idx(0, 0)
        wait_idx(0, 0)
        start_gather(0, 0)
        start_idx(1, 1)
        wait_idx(1, 1)
        start_gather(1, 1)
        wait_gather(0)
        start_store(0, 0)
        start_idx(2, 0)

        def body(j, carry):
            i0 = 2 * j
            i1 = i0 + 1
            wait_store(i0 - 2, 0)
            wait_idx(i0, 0)
            start_gather(i0, 0)
            wait_gather(1)
            start_store(i0 - 1, 1)
            start_idx(i1, 1)
            wait_store(i1 - 2, 1)
            wait_idx(i1, 1)
            start_gather(i1, 1)
            wait_gather(0)
            start_store(i0, 0)
            start_idx(i0 + 2, 0)
            return carry

        lax.fori_loop(1, n_chunks // 2, body, 0)

        wait_gather(1)
        start_store(n_chunks - 1, 1)
        wait_idx(n_chunks, 0)
        wait_store(n_chunks - 2, 0)
        wait_store(n_chunks - 1, 1)

    return lookup


def kernel(idx, x, table):
    del x
    b, l = idx.shape
    n = b * l
    idx_flat = idx.reshape(n).astype(jnp.int32)
    lookup = _make_lookup(n, table.shape[0], table.shape[1])
    out = lookup(table.astype(jnp.float32), idx_flat)
    return out.reshape(b, l, table.shape[1])
